# baseline (speedup 1.0000x reference)
import functools

import jax
import jax.numpy as jnp
from jax import lax
from jax.experimental import pallas as pl
from jax.experimental.pallas import tpu as pltpu
from jax.experimental.pallas import tpu_sc as plsc

_NC = 2
_NS = 16
_NW = _NC * _NS

_CH = 128

_TB = 2048


def _transpose_pack_body(x_ref, o_ref):
    xt = x_ref[...].T
    o_ref[:, :64] = xt[: _TB // 2]
    o_ref[:, 64:] = xt[_TB // 2 :]


def _select_relu_linear_body(x_ref, h_ref, wt_ref, b_ref, o_ref):
    x = x_ref[...]
    sel = jnp.where(h_ref[...] == 1, x[:, 64:], x[:, :64])
    sel = jnp.maximum(sel, 0.0)
    o_ref[...] = (
        jnp.dot(sel, wt_ref[...], preferred_element_type=jnp.float32)
        + b_ref[...]
    )


@functools.partial(jax.jit, static_argnames=("bpw",))
def _sc_gather(pidx, t2, *, bpw):
    B = _NW * bpw
    nch = bpw // _CH
    mesh = plsc.VectorSubcoreMesh(core_axis_name="c", subcore_axis_name="s")

    @functools.partial(
        pl.kernel,
        mesh=mesh,
        out_type=jax.ShapeDtypeStruct((B, 128), jnp.float32),
        scratch_types=[
            pltpu.VMEM((nch, _CH), jnp.int32),
            pltpu.VMEM((bpw, 128), jnp.float32),
            pltpu.SemaphoreType.DMA,
        ],
    )
    def gather_kernel(idx_hbm, tab_hbm, out_hbm, idx_v, rows_v, sem):
        wid = lax.axis_index("s") * _NC + lax.axis_index("c")
        base = wid * bpw
        pltpu.sync_copy(idx_hbm.at[wid], idx_v)
        copies = [
            pltpu.async_copy(
                tab_hbm.at[idx_v.at[j]],
                rows_v.at[pl.ds(j * _CH, _CH)],
                sem,
            )
            for j in range(nch)
        ]
        for c in copies:
            c.wait()
        pltpu.sync_copy(rows_v, out_hbm.at[pl.ds(base, bpw)])

    return gather_kernel(pidx, t2)


def kernel(user_idx, embedding_table, linear_w, linear_b):
    B = user_idx.shape[0]
    V, D = embedding_table.shape
    F = linear_w.shape[0]

    bpw = B // _NW
    nch = bpw // _CH
    idx = user_idx.reshape(-1).astype(jnp.int32)
    pidx = ((idx >> 11) * (_TB // 2) + (idx & (_TB // 2 - 1))).reshape(
        _NW, nch, _CH
    )
    half = ((idx >> 10) & 1).reshape(B, 1)

    nblk = (V + _TB - 1) // _TB
    t2 = pl.pallas_call(
        _transpose_pack_body,
        grid=(nblk,),
        in_specs=[pl.BlockSpec((D, _TB), lambda i: (0, i))],
        out_specs=pl.BlockSpec((_TB // 2, 2 * D), lambda i: (i, 0)),
        out_shape=jax.ShapeDtypeStruct((nblk * (_TB // 2), 2 * D), jnp.float32),
    )(embedding_table.T)

    gathered = _sc_gather(pidx, t2, bpw=bpw)

    BM = 1024
    out2d = pl.pallas_call(
        _select_relu_linear_body,
        grid=(B // BM,),
        in_specs=[
            pl.BlockSpec((BM, 2 * D), lambda i: (i, 0)),
            pl.BlockSpec((BM, 1), lambda i: (i, 0)),
            pl.BlockSpec((D, F), lambda i: (0, 0)),
            pl.BlockSpec((1, F), lambda i: (0, 0)),
        ],
        out_specs=pl.BlockSpec((BM, F), lambda i: (i, 0)),
        out_shape=jax.ShapeDtypeStruct((B, F), jnp.float32),
    )(gathered, half, linear_w.T, linear_b.reshape(1, F))

    return out2d.reshape(B, 1, F)


# speedup vs baseline: 1.3000x; 1.3000x over previous
import functools

import jax
import jax.numpy as jnp
from jax import lax
from jax.experimental import pallas as pl
from jax.experimental.pallas import tpu as pltpu
from jax.experimental.pallas import tpu_sc as plsc

_NC = 2
_NS = 16
_NW = _NC * _NS

_CH = 128

_TB = 4096


def _transpose_pack_body(x_ref, eye_ref, o_ref):
    xt = lax.dot_general(
        x_ref[...],
        eye_ref[...],
        dimension_numbers=(((0,), (0,)), ((), ())),
        preferred_element_type=jnp.float32,
    )
    o_ref[:, :64] = xt[: _TB // 2]
    o_ref[:, 64:] = xt[_TB // 2 :]


def _select_relu_linear_body(x_ref, h_ref, wt_ref, b_ref, o_ref):
    x = x_ref[...]
    sel = jnp.where(h_ref[...] == 1, x[:, 64:], x[:, :64])
    sel = jnp.maximum(sel, 0.0)
    o_ref[...] = (
        jnp.dot(sel, wt_ref[...], preferred_element_type=jnp.float32)
        + b_ref[...]
    )


@functools.partial(jax.jit, static_argnames=("bpw",))
def _sc_gather(pidx, t2, *, bpw):
    B = _NW * bpw
    nch = bpw // _CH
    mesh = plsc.VectorSubcoreMesh(core_axis_name="c", subcore_axis_name="s")

    @functools.partial(
        pl.kernel,
        mesh=mesh,
        out_type=jax.ShapeDtypeStruct((B, 128), jnp.float32),
        scratch_types=[
            pltpu.VMEM((nch, _CH), jnp.int32),
            pltpu.VMEM((bpw, 128), jnp.float32),
            pltpu.SemaphoreType.DMA,
        ],
    )
    def gather_kernel(idx_hbm, tab_hbm, out_hbm, idx_v, rows_v, sem):
        wid = lax.axis_index("s") * _NC + lax.axis_index("c")
        base = wid * bpw
        pltpu.sync_copy(idx_hbm.at[wid], idx_v)
        copies = [
            pltpu.async_copy(
                tab_hbm.at[idx_v.at[j]],
                rows_v.at[pl.ds(j * _CH, _CH)],
                sem,
            )
            for j in range(nch)
        ]
        for c in copies:
            c.wait()
        pltpu.sync_copy(rows_v, out_hbm.at[pl.ds(base, bpw)])

    return gather_kernel(pidx, t2)


def kernel(user_idx, embedding_table, linear_w, linear_b):
    B = user_idx.shape[0]
    V, D = embedding_table.shape
    F = linear_w.shape[0]

    bpw = B // _NW
    nch = bpw // _CH
    idx = user_idx.reshape(-1).astype(jnp.int32)
    hb = _TB // 2
    pidx = ((idx // _TB) * hb + (idx & (hb - 1))).reshape(_NW, nch, _CH)
    half = ((idx // hb) & 1).reshape(B, 1)

    nblk = (V + _TB - 1) // _TB
    t2 = pl.pallas_call(
        _transpose_pack_body,
        grid=(nblk,),
        in_specs=[
            pl.BlockSpec((D, _TB), lambda i: (0, i)),
            pl.BlockSpec((D, D), lambda i: (0, 0)),
        ],
        out_specs=pl.BlockSpec((_TB // 2, 2 * D), lambda i: (i, 0)),
        out_shape=jax.ShapeDtypeStruct((nblk * (_TB // 2), 2 * D), jnp.float32),
    )(embedding_table.T, jnp.eye(D, dtype=jnp.float32))

    gathered = _sc_gather(pidx, t2, bpw=bpw)

    BM = 1024
    out2d = pl.pallas_call(
        _select_relu_linear_body,
        grid=(B // BM,),
        in_specs=[
            pl.BlockSpec((BM, 2 * D), lambda i: (i, 0)),
            pl.BlockSpec((BM, 1), lambda i: (i, 0)),
            pl.BlockSpec((D, F), lambda i: (0, 0)),
            pl.BlockSpec((1, F), lambda i: (0, 0)),
        ],
        out_specs=pl.BlockSpec((BM, F), lambda i: (i, 0)),
        out_shape=jax.ShapeDtypeStruct((B, F), jnp.float32),
    )(gathered, half, linear_w.T, linear_b.reshape(1, F))

    return out2d.reshape(B, 1, F)


# speedup vs baseline: 1.5742x; 1.2109x over previous
import functools

import jax
import jax.numpy as jnp
from jax import lax
from jax.experimental import pallas as pl
from jax.experimental.pallas import tpu as pltpu
from jax.experimental.pallas import tpu_sc as plsc

_NC = 2
_NS = 16
_NW = _NC * _NS

_CH = 128

_TB = 8192


def _transpose_pack_body(x_ref, eye_ref, o_ref):
    xt = lax.dot_general(
        x_ref[...],
        eye_ref[...],
        dimension_numbers=(((0,), (0,)), ((), ())),
        preferred_element_type=jnp.float32,
    )
    o_ref[:, :64] = xt[: _TB // 2]
    o_ref[:, 64:] = xt[_TB // 2 :]


def _select_relu_linear_body(x_ref, h_ref, wt_ref, b_ref, o_ref):
    x = x_ref[...]
    sel = jnp.where(h_ref[...] == 1, x[:, 64:], x[:, :64])
    sel = jnp.maximum(sel, 0.0)
    o_ref[...] = (
        jnp.dot(sel, wt_ref[...], preferred_element_type=jnp.float32)
        + b_ref[...]
    )


@functools.partial(jax.jit, static_argnames=("bpw",))
def _sc_gather(pidx, t2, *, bpw):
    B = _NW * bpw
    nch = bpw // _CH
    mesh = plsc.VectorSubcoreMesh(core_axis_name="c", subcore_axis_name="s")

    @functools.partial(
        pl.kernel,
        mesh=mesh,
        out_type=jax.ShapeDtypeStruct((B, 128), jnp.float32),
        scratch_types=[
            pltpu.VMEM((nch, _CH), jnp.int32),
            pltpu.VMEM((bpw, 128), jnp.float32),
            pltpu.SemaphoreType.DMA,
        ],
    )
    def gather_kernel(idx_hbm, tab_hbm, out_hbm, idx_v, rows_v, sem):
        wid = lax.axis_index("s") * _NC + lax.axis_index("c")
        base = wid * bpw
        pltpu.sync_copy(idx_hbm.at[wid], idx_v)
        copies = [
            pltpu.async_copy(
                tab_hbm.at[idx_v.at[j]],
                rows_v.at[pl.ds(j * _CH, _CH)],
                sem,
            )
            for j in range(nch)
        ]
        for c in copies:
            c.wait()
        pltpu.sync_copy(rows_v, out_hbm.at[pl.ds(base, bpw)])

    return gather_kernel(pidx, t2)


def kernel(user_idx, embedding_table, linear_w, linear_b):
    B = user_idx.shape[0]
    V, D = embedding_table.shape
    F = linear_w.shape[0]

    bpw = B // _NW
    nch = bpw // _CH
    idx = user_idx.reshape(-1).astype(jnp.int32)
    hb = _TB // 2
    pidx = ((idx // _TB) * hb + (idx & (hb - 1))).reshape(_NW, nch, _CH)
    half = ((idx // hb) & 1).reshape(B, 1)

    nblk = (V + _TB - 1) // _TB
    t2 = pl.pallas_call(
        _transpose_pack_body,
        grid=(nblk,),
        in_specs=[
            pl.BlockSpec((D, _TB), lambda i: (0, i)),
            pl.BlockSpec((D, D), lambda i: (0, 0)),
        ],
        out_specs=pl.BlockSpec((_TB // 2, 2 * D), lambda i: (i, 0)),
        out_shape=jax.ShapeDtypeStruct((nblk * (_TB // 2), 2 * D), jnp.float32),
    )(embedding_table.T, jnp.eye(D, dtype=jnp.float32))

    gathered = _sc_gather(pidx, t2, bpw=bpw)

    BM = 1024
    out2d = pl.pallas_call(
        _select_relu_linear_body,
        grid=(B // BM,),
        in_specs=[
            pl.BlockSpec((BM, 2 * D), lambda i: (i, 0)),
            pl.BlockSpec((BM, 1), lambda i: (i, 0)),
            pl.BlockSpec((D, F), lambda i: (0, 0)),
            pl.BlockSpec((1, F), lambda i: (0, 0)),
        ],
        out_specs=pl.BlockSpec((BM, F), lambda i: (i, 0)),
        out_shape=jax.ShapeDtypeStruct((B, F), jnp.float32),
    )(gathered, half, linear_w.T, linear_b.reshape(1, F))

    return out2d.reshape(B, 1, F)


# speedup vs baseline: 1.7493x; 1.1113x over previous
import functools

import jax
import jax.numpy as jnp
from jax import lax
from jax.experimental import pallas as pl
from jax.experimental.pallas import tpu as pltpu
from jax.experimental.pallas import tpu_sc as plsc

_NC = 2
_NS = 16
_NW = _NC * _NS

_CH = 128

_TB = 16384


def _transpose_pack_body(x_ref, eye_ref, o_ref):
    xt = lax.dot_general(
        x_ref[...],
        eye_ref[...],
        dimension_numbers=(((0,), (0,)), ((), ())),
        preferred_element_type=jnp.float32,
    )
    o_ref[:, :64] = xt[: _TB // 2]
    o_ref[:, 64:] = xt[_TB // 2 :]


def _select_relu_linear_body(x_ref, h_ref, wt_ref, b_ref, o_ref):
    x = x_ref[...]
    sel = jnp.where(h_ref[...] == 1, x[:, 64:], x[:, :64])
    sel = jnp.maximum(sel, 0.0)
    o_ref[...] = (
        jnp.dot(sel, wt_ref[...], preferred_element_type=jnp.float32)
        + b_ref[...]
    )


@functools.partial(jax.jit, static_argnames=("bpw",))
def _sc_gather(pidx, t2, *, bpw):
    B = _NW * bpw
    nch = bpw // _CH
    mesh = plsc.VectorSubcoreMesh(core_axis_name="c", subcore_axis_name="s")

    @functools.partial(
        pl.kernel,
        mesh=mesh,
        out_type=jax.ShapeDtypeStruct((B, 128), jnp.float32),
        scratch_types=[
            pltpu.VMEM((nch, _CH), jnp.int32),
            pltpu.VMEM((bpw, 128), jnp.float32),
            pltpu.SemaphoreType.DMA,
        ],
    )
    def gather_kernel(idx_hbm, tab_hbm, out_hbm, idx_v, rows_v, sem):
        wid = lax.axis_index("s") * _NC + lax.axis_index("c")
        base = wid * bpw
        pltpu.sync_copy(idx_hbm.at[wid], idx_v)
        copies = [
            pltpu.async_copy(
                tab_hbm.at[idx_v.at[j]],
                rows_v.at[pl.ds(j * _CH, _CH)],
                sem,
            )
            for j in range(nch)
        ]
        for c in copies:
            c.wait()
        pltpu.sync_copy(rows_v, out_hbm.at[pl.ds(base, bpw)])

    return gather_kernel(pidx, t2)


def kernel(user_idx, embedding_table, linear_w, linear_b):
    B = user_idx.shape[0]
    V, D = embedding_table.shape
    F = linear_w.shape[0]

    bpw = B // _NW
    nch = bpw // _CH
    idx = user_idx.reshape(-1).astype(jnp.int32)
    hb = _TB // 2
    pidx = ((idx // _TB) * hb + (idx & (hb - 1))).reshape(_NW, nch, _CH)
    half = ((idx // hb) & 1).reshape(B, 1)

    nblk = (V + _TB - 1) // _TB
    t2 = pl.pallas_call(
        _transpose_pack_body,
        grid=(nblk,),
        in_specs=[
            pl.BlockSpec((D, _TB), lambda i: (0, i)),
            pl.BlockSpec((D, D), lambda i: (0, 0)),
        ],
        out_specs=pl.BlockSpec((_TB // 2, 2 * D), lambda i: (i, 0)),
        out_shape=jax.ShapeDtypeStruct((nblk * (_TB // 2), 2 * D), jnp.float32),
    )(embedding_table.T, jnp.eye(D, dtype=jnp.float32))

    gathered = _sc_gather(pidx, t2, bpw=bpw)

    BM = 1024
    out2d = pl.pallas_call(
        _select_relu_linear_body,
        grid=(B // BM,),
        in_specs=[
            pl.BlockSpec((BM, 2 * D), lambda i: (i, 0)),
            pl.BlockSpec((BM, 1), lambda i: (i, 0)),
            pl.BlockSpec((D, F), lambda i: (0, 0)),
            pl.BlockSpec((1, F), lambda i: (0, 0)),
        ],
        out_specs=pl.BlockSpec((BM, F), lambda i: (i, 0)),
        out_shape=jax.ShapeDtypeStruct((B, F), jnp.float32),
    )(gathered, half, linear_w.T, linear_b.reshape(1, F))

    return out2d.reshape(B, 1, F)


# speedup vs baseline: 1.8461x; 1.0553x over previous
import functools

import jax
import jax.numpy as jnp
from jax import lax
from jax.experimental import pallas as pl
from jax.experimental.pallas import tpu as pltpu
from jax.experimental.pallas import tpu_sc as plsc

_NC = 2
_NS = 16
_NW = _NC * _NS

_CH = 128

_TB = 32768


def _transpose_pack_body(x_ref, eye_ref, o_ref):
    xt = lax.dot_general(
        x_ref[...],
        eye_ref[...],
        dimension_numbers=(((0,), (0,)), ((), ())),
        preferred_element_type=jnp.float32,
    )
    o_ref[:, :64] = xt[: _TB // 2]
    o_ref[:, 64:] = xt[_TB // 2 :]


def _select_relu_linear_body(x_ref, h_ref, wt_ref, b_ref, o_ref):
    x = x_ref[...]
    sel = jnp.where(h_ref[...] == 1, x[:, 64:], x[:, :64])
    sel = jnp.maximum(sel, 0.0)
    o_ref[...] = (
        jnp.dot(sel, wt_ref[...], preferred_element_type=jnp.float32)
        + b_ref[...]
    )


@functools.partial(jax.jit, static_argnames=("bpw",))
def _sc_gather(pidx, t2, *, bpw):
    B = _NW * bpw
    nch = bpw // _CH
    mesh = plsc.VectorSubcoreMesh(core_axis_name="c", subcore_axis_name="s")

    @functools.partial(
        pl.kernel,
        mesh=mesh,
        out_type=jax.ShapeDtypeStruct((B, 128), jnp.float32),
        scratch_types=[
            pltpu.VMEM((nch, _CH), jnp.int32),
            pltpu.VMEM((bpw, 128), jnp.float32),
            pltpu.SemaphoreType.DMA,
        ],
    )
    def gather_kernel(idx_hbm, tab_hbm, out_hbm, idx_v, rows_v, sem):
        wid = lax.axis_index("s") * _NC + lax.axis_index("c")
        base = wid * bpw
        pltpu.sync_copy(idx_hbm.at[wid], idx_v)
        copies = [
            pltpu.async_copy(
                tab_hbm.at[idx_v.at[j]],
                rows_v.at[pl.ds(j * _CH, _CH)],
                sem,
            )
            for j in range(nch)
        ]
        for c in copies:
            c.wait()
        pltpu.sync_copy(rows_v, out_hbm.at[pl.ds(base, bpw)])

    return gather_kernel(pidx, t2)


def kernel(user_idx, embedding_table, linear_w, linear_b):
    B = user_idx.shape[0]
    V, D = embedding_table.shape
    F = linear_w.shape[0]

    bpw = B // _NW
    nch = bpw // _CH
    idx = user_idx.reshape(-1).astype(jnp.int32)
    hb = _TB // 2
    pidx = ((idx // _TB) * hb + (idx & (hb - 1))).reshape(_NW, nch, _CH)
    half = ((idx // hb) & 1).reshape(B, 1)

    nblk = (V + _TB - 1) // _TB
    t2 = pl.pallas_call(
        _transpose_pack_body,
        grid=(nblk,),
        in_specs=[
            pl.BlockSpec((D, _TB), lambda i: (0, i)),
            pl.BlockSpec((D, D), lambda i: (0, 0)),
        ],
        out_specs=pl.BlockSpec((_TB // 2, 2 * D), lambda i: (i, 0)),
        out_shape=jax.ShapeDtypeStruct((nblk * (_TB // 2), 2 * D), jnp.float32),
    )(embedding_table.T, jnp.eye(D, dtype=jnp.float32))

    gathered = _sc_gather(pidx, t2, bpw=bpw)

    BM = 1024
    out2d = pl.pallas_call(
        _select_relu_linear_body,
        grid=(B // BM,),
        in_specs=[
            pl.BlockSpec((BM, 2 * D), lambda i: (i, 0)),
            pl.BlockSpec((BM, 1), lambda i: (i, 0)),
            pl.BlockSpec((D, F), lambda i: (0, 0)),
            pl.BlockSpec((1, F), lambda i: (0, 0)),
        ],
        out_specs=pl.BlockSpec((BM, F), lambda i: (i, 0)),
        out_shape=jax.ShapeDtypeStruct((B, F), jnp.float32),
    )(gathered, half, linear_w.T, linear_b.reshape(1, F))

    return out2d.reshape(B, 1, F)


# speedup vs baseline: 1.9036x; 1.0311x over previous
import functools

import jax
import jax.numpy as jnp
from jax import lax
from jax.experimental import pallas as pl
from jax.experimental.pallas import tpu as pltpu
from jax.experimental.pallas import tpu_sc as plsc

_NC = 2
_NS = 16
_NW = _NC * _NS

_CH = 128

_TB = 32768


def _transpose_pack_body(x_ref, eye_ref, o_ref):
    xt = lax.dot_general(
        x_ref[...],
        eye_ref[...],
        dimension_numbers=(((0,), (0,)), ((), ())),
        preferred_element_type=jnp.float32,
    )
    o_ref[:, :64] = xt[: _TB // 2]
    o_ref[:, 64:] = xt[_TB // 2 :]


def _select_relu_linear_body(x_ref, h_ref, w_ref, b_ref, o_ref):
    x = x_ref[...]
    sel = jnp.where(h_ref[...] == 1, x[:, 64:], x[:, :64])
    sel = jnp.maximum(sel, 0.0)
    o_ref[...] = (
        lax.dot_general(
            w_ref[...],
            sel,
            dimension_numbers=(((1,), (1,)), ((), ())),
            preferred_element_type=jnp.float32,
        )
        + b_ref[...]
    )


@functools.partial(jax.jit, static_argnames=("bpw",))
def _sc_gather(pidx, t2, *, bpw):
    B = _NW * bpw
    nch = bpw // _CH
    mesh = plsc.VectorSubcoreMesh(core_axis_name="c", subcore_axis_name="s")

    @functools.partial(
        pl.kernel,
        mesh=mesh,
        out_type=jax.ShapeDtypeStruct((B, 128), jnp.float32),
        scratch_types=[
            pltpu.VMEM((nch, _CH), jnp.int32),
            pltpu.VMEM((bpw, 128), jnp.float32),
            pltpu.SemaphoreType.DMA,
        ],
    )
    def gather_kernel(idx_hbm, tab_hbm, out_hbm, idx_v, rows_v, sem):
        wid = lax.axis_index("s") * _NC + lax.axis_index("c")
        base = wid * bpw
        pltpu.sync_copy(idx_hbm.at[wid], idx_v)
        copies = [
            pltpu.async_copy(
                tab_hbm.at[idx_v.at[j]],
                rows_v.at[pl.ds(j * _CH, _CH)],
                sem,
            )
            for j in range(nch)
        ]
        for c in copies:
            c.wait()
        pltpu.sync_copy(rows_v, out_hbm.at[pl.ds(base, bpw)])

    return gather_kernel(pidx, t2)


def kernel(user_idx, embedding_table, linear_w, linear_b):
    B = user_idx.shape[0]
    V, D = embedding_table.shape
    F = linear_w.shape[0]

    bpw = B // _NW
    nch = bpw // _CH
    idx = user_idx.reshape(-1).astype(jnp.int32)
    hb = _TB // 2
    pidx = ((idx // _TB) * hb + (idx & (hb - 1))).reshape(_NW, nch, _CH)
    half = ((idx // hb) & 1).reshape(B, 1)

    nblk = (V + _TB - 1) // _TB
    t2 = pl.pallas_call(
        _transpose_pack_body,
        grid=(nblk,),
        in_specs=[
            pl.BlockSpec((D, _TB), lambda i: (0, i)),
            pl.BlockSpec((D, D), lambda i: (0, 0)),
        ],
        out_specs=pl.BlockSpec((_TB // 2, 2 * D), lambda i: (i, 0)),
        out_shape=jax.ShapeDtypeStruct((nblk * (_TB // 2), 2 * D), jnp.float32),
    )(embedding_table.T, jnp.eye(D, dtype=jnp.float32))

    gathered = _sc_gather(pidx, t2, bpw=bpw)

    BM = 1024
    out_t = pl.pallas_call(
        _select_relu_linear_body,
        grid=(B // BM,),
        in_specs=[
            pl.BlockSpec((BM, 2 * D), lambda i: (i, 0)),
            pl.BlockSpec((BM, 1), lambda i: (i, 0)),
            pl.BlockSpec((F, D), lambda i: (0, 0)),
            pl.BlockSpec((F, 1), lambda i: (0, 0)),
        ],
        out_specs=pl.BlockSpec((F, BM), lambda i: (0, i)),
        out_shape=jax.ShapeDtypeStruct((F, B), jnp.float32),
    )(gathered, half, linear_w, linear_b.reshape(F, 1))

    return out_t.T.reshape(B, 1, F)
